# baseline (speedup 1.0000x reference)
import functools

import jax
import jax.numpy as jnp
from jax import lax
from jax.experimental import pallas as pl
from jax.experimental.pallas import tpu as pltpu
from jax.experimental.pallas import tpu_sc as plsc

_NW = 32
_L = 16
_ROUND_C = float(1.5 * 2.0 ** 23)
_TPAD = 80


def _sc_body(x_hbm, s_hbm, m_hbm, tab_hbm, cst_hbm, out_hbm,
             tab_v, cst_v, xb, sb, mb, ob):
    n = x_hbm.shape[0]
    per_w = n // _NW
    ch = xb.shape[0]
    nch = per_w // ch
    wid = lax.axis_index("s") * 2 + lax.axis_index("c")

    pltpu.sync_copy(tab_hbm, tab_v)
    pltpu.sync_copy(cst_hbm, cst_v)
    av = cst_v[0:_L]
    bv = cst_v[_L:2 * _L]

    def chunk(c, carry):
        base = wid * per_w + c * ch
        pltpu.sync_copy(x_hbm.at[pl.ds(base, ch)], xb)
        pltpu.sync_copy(s_hbm.at[pl.ds(base, ch)], sb)
        pltpu.sync_copy(m_hbm.at[pl.ds(base, ch)], mb)

        def vec(i, carry2):
            off = pl.multiple_of(i * _L, _L)
            xv = xb[pl.ds(off, _L)]
            sv = sb[pl.ds(off, _L)]
            mv = mb[pl.ds(off, _L)]
            sa = jnp.abs(sv)
            bits = lax.bitcast_convert_type(sa, jnp.int32)
            u = bits.astype(jnp.float32) * av + bv
            w = jnp.clip(u.astype(jnp.int32) - 1, 0, 62)
            t0 = plsc.load_gather(tab_v, [w])
            t1 = plsc.load_gather(tab_v, [w + 1])
            t2 = plsc.load_gather(tab_v, [w + 2])
            t3 = plsc.load_gather(tab_v, [w + 3])
            d0 = sa - t0
            d1 = sa - t1
            d2 = sa - t2
            d3 = sa - t3
            qs = jnp.where(d0 + d1 > 0.0, t1, t0)
            qs = jnp.where(d1 + d2 > 0.0, t2, qs)
            qs = jnp.where(d2 + d3 > 0.0, t3, qs)
            v = (xv - mv) / qs
            r = (v + _ROUND_C) - _ROUND_C
            ob[pl.ds(off, _L)] = r * qs + mv
            return carry2

        lax.fori_loop(0, ch // _L, vec, 0)
        pltpu.sync_copy(ob, out_hbm.at[pl.ds(base, ch)])
        return carry

    lax.fori_loop(0, nch, chunk, 0)


@jax.jit
def kernel(inputs, scale, mean, scale_table):
    B, H, W = inputs.shape
    n = B * H * W
    x = inputs.reshape(n)
    s = scale.reshape(n)
    m = mean.reshape(n)

    tab = jnp.concatenate(
        [scale_table, jnp.full((_TPAD - 64,), 1e30, jnp.float32)])
    t0 = scale_table[0]
    t63 = scale_table[63]
    dlog2 = (jnp.log2(t63) - jnp.log2(t0)) * jnp.float32(1.0 / 63.0)
    a = jnp.float32(1.0 / 8388608.0) / dlog2
    b = -(jnp.float32(127.0430357) + jnp.log2(t0)) / dlog2
    cst = jnp.concatenate([jnp.full((_L,), a, jnp.float32),
                           jnp.full((_L,), b, jnp.float32)])

    ch = 4608
    mesh = plsc.VectorSubcoreMesh(core_axis_name="c", subcore_axis_name="s")
    fn = functools.partial(
        pl.kernel,
        mesh=mesh,
        out_type=jax.ShapeDtypeStruct((n,), jnp.float32),
        compiler_params=pltpu.CompilerParams(needs_layout_passes=False),
        scratch_types=[
            pltpu.VMEM((_TPAD,), jnp.float32),
            pltpu.VMEM((2 * _L,), jnp.float32),
            pltpu.VMEM((ch,), jnp.float32),
            pltpu.VMEM((ch,), jnp.float32),
            pltpu.VMEM((ch,), jnp.float32),
            pltpu.VMEM((ch,), jnp.float32),
        ],
    )(_sc_body)
    out = fn(x, s, m, tab, cst)
    return out.reshape(B, H, W)


# speedup vs baseline: 1.6854x; 1.6854x over previous
import functools

import jax
import jax.numpy as jnp
from jax import lax
from jax.experimental import pallas as pl
from jax.experimental.pallas import tpu as pltpu
from jax.experimental.pallas import tpu_sc as plsc

_NW = 32
_L = 16
_ROUND_C = float(1.5 * 2.0 ** 23)
_TPAD = 80
_CH = 4608
_NCH = 8


def _sc_body(x_hbm, s_hbm, m_hbm, tab_hbm, cst_hbm, out_hbm,
             tab_v, cst_v, xb0, xb1, sb0, sb1, mb0, mb1, ob0, ob1,
             isem0, isem1, osem0, osem1):
    per_w = x_hbm.shape[0] // _NW
    wid = lax.axis_index("s") * 2 + lax.axis_index("c")

    pltpu.sync_copy(tab_hbm, tab_v)
    pltpu.sync_copy(cst_hbm, cst_v)
    av = cst_v[0:_L]
    bv = cst_v[_L:2 * _L]

    xbufs, sbufs, mbufs, obufs = (xb0, xb1), (sb0, sb1), (mb0, mb1), (ob0, ob1)
    isems, osems = (isem0, isem1), (osem0, osem1)

    def sl(c):
        return pl.ds(wid * per_w + c * _CH, _CH)

    def fire_in(c):
        b = c % 2
        return (pltpu.async_copy(x_hbm.at[sl(c)], xbufs[b], isems[b]),
                pltpu.async_copy(s_hbm.at[sl(c)], sbufs[b], isems[b]),
                pltpu.async_copy(m_hbm.at[sl(c)], mbufs[b], isems[b]))

    hin = fire_in(0)
    hout = [None, None]
    for c in range(_NCH):
        b = c % 2
        nxt = fire_in(c + 1) if c + 1 < _NCH else None
        for h in hin:
            h.wait()
        if hout[b] is not None:
            hout[b].wait()
        xb, sb, mb, ob = xbufs[b], sbufs[b], mbufs[b], obufs[b]

        @plsc.parallel_loop(0, _CH, _L, unroll=8)
        def vec(off):
            off = pl.multiple_of(off, _L)
            xv = xb[pl.ds(off, _L)]
            sv = sb[pl.ds(off, _L)]
            mv = mb[pl.ds(off, _L)]
            sa = jnp.abs(sv)
            bits = lax.bitcast_convert_type(sa, jnp.int32)
            u = bits.astype(jnp.float32) * av + bv
            w = jnp.clip(u.astype(jnp.int32) - 1, 0, 62)
            t0 = plsc.load_gather(tab_v, [w])
            t1 = plsc.load_gather(tab_v, [w + 1])
            t2 = plsc.load_gather(tab_v, [w + 2])
            t3 = plsc.load_gather(tab_v, [w + 3])
            d0 = sa - t0
            d1 = sa - t1
            d2 = sa - t2
            d3 = sa - t3
            qs = jnp.where(d0 + d1 > 0.0, t1, t0)
            qs = jnp.where(d1 + d2 > 0.0, t2, qs)
            qs = jnp.where(d2 + d3 > 0.0, t3, qs)
            v = (xv - mv) / qs
            r = (v + _ROUND_C) - _ROUND_C
            ob[pl.ds(off, _L)] = r * qs + mv

        hout[b] = pltpu.async_copy(ob, out_hbm.at[sl(c)], osems[b])
        hin = nxt
    for h in hout:
        if h is not None:
            h.wait()


@jax.jit
def kernel(inputs, scale, mean, scale_table):
    B, H, W = inputs.shape
    n = B * H * W
    x = inputs.reshape(n)
    s = scale.reshape(n)
    m = mean.reshape(n)

    tab = jnp.concatenate(
        [scale_table, jnp.full((_TPAD - 64,), 1e30, jnp.float32)])
    t0 = scale_table[0]
    t63 = scale_table[63]
    dlog2 = (jnp.log2(t63) - jnp.log2(t0)) * jnp.float32(1.0 / 63.0)
    a = jnp.float32(1.0 / 8388608.0) / dlog2
    b = -(jnp.float32(127.0430357) + jnp.log2(t0)) / dlog2
    cst = jnp.concatenate([jnp.full((_L,), a, jnp.float32),
                           jnp.full((_L,), b, jnp.float32)])

    mesh = plsc.VectorSubcoreMesh(core_axis_name="c", subcore_axis_name="s")
    fn = functools.partial(
        pl.kernel,
        mesh=mesh,
        out_type=jax.ShapeDtypeStruct((n,), jnp.float32),
        compiler_params=pltpu.CompilerParams(needs_layout_passes=False),
        scratch_types=[
            pltpu.VMEM((_TPAD,), jnp.float32),
            pltpu.VMEM((2 * _L,), jnp.float32),
            pltpu.VMEM((_CH,), jnp.float32),
            pltpu.VMEM((_CH,), jnp.float32),
            pltpu.VMEM((_CH,), jnp.float32),
            pltpu.VMEM((_CH,), jnp.float32),
            pltpu.VMEM((_CH,), jnp.float32),
            pltpu.VMEM((_CH,), jnp.float32),
            pltpu.VMEM((_CH,), jnp.float32),
            pltpu.VMEM((_CH,), jnp.float32),
            pltpu.SemaphoreType.DMA,
            pltpu.SemaphoreType.DMA,
            pltpu.SemaphoreType.DMA,
            pltpu.SemaphoreType.DMA,
        ],
    )(_sc_body)
    out = fn(x, s, m, tab, cst)
    return out.reshape(B, H, W)


# speedup vs baseline: 1.8263x; 1.0836x over previous
import functools

import jax
import jax.numpy as jnp
from jax import lax
from jax.experimental import pallas as pl
from jax.experimental.pallas import tpu as pltpu
from jax.experimental.pallas import tpu_sc as plsc

_NW = 32
_L = 16
_ROUND_C = float(1.5 * 2.0 ** 23)
_TPAD = 80
_CH = 4608
_NCH = 8


def _sc_body(x_hbm, s_hbm, m_hbm, tab_hbm, cst_hbm, out_hbm,
             tab_v, cst_v, xb0, xb1, sb0, sb1, mb0, mb1, ob0, ob1,
             isem0, isem1, osem0, osem1):
    per_w = x_hbm.shape[0] // _NW
    wid = lax.axis_index("s") * 2 + lax.axis_index("c")

    pltpu.sync_copy(tab_hbm, tab_v)
    pltpu.sync_copy(cst_hbm, cst_v)
    av = cst_v[0:_L]
    bv = cst_v[_L:2 * _L]

    xbufs, sbufs, mbufs, obufs = (xb0, xb1), (sb0, sb1), (mb0, mb1), (ob0, ob1)
    isems, osems = (isem0, isem1), (osem0, osem1)

    def sl(c):
        return pl.ds(wid * per_w + c * _CH, _CH)

    def fire_in(c):
        b = c % 2
        return (pltpu.async_copy(x_hbm.at[sl(c)], xbufs[b], isems[b]),
                pltpu.async_copy(s_hbm.at[sl(c)], sbufs[b], isems[b]),
                pltpu.async_copy(m_hbm.at[sl(c)], mbufs[b], isems[b]))

    hin = fire_in(0)
    hout = [None, None]
    for c in range(_NCH):
        b = c % 2
        nxt = fire_in(c + 1) if c + 1 < _NCH else None
        for h in hin:
            h.wait()
        if hout[b] is not None:
            hout[b].wait()
        xb, sb, mb, ob = xbufs[b], sbufs[b], mbufs[b], obufs[b]

        @plsc.parallel_loop(0, _CH, _L, unroll=8)
        def vec(off):
            off = pl.multiple_of(off, _L)
            xv = xb[pl.ds(off, _L)]
            sv = sb[pl.ds(off, _L)]
            mv = mb[pl.ds(off, _L)]
            sa = jnp.abs(sv)
            bits = lax.bitcast_convert_type(sa, jnp.int32)
            u = bits.astype(jnp.float32) * av + bv
            w = jnp.clip(u.astype(jnp.int32) - 1, 0, 62)
            t0 = plsc.load_gather(tab_v, [w])
            t1 = plsc.load_gather(tab_v, [w + 1])
            t2 = plsc.load_gather(tab_v, [w + 2])
            d0 = sa - t0
            d1 = sa - t1
            d2 = sa - t2
            qs = jnp.where(d0 + d1 > 0.0, t1, t0)
            qs = jnp.where(d1 + d2 > 0.0, t2, qs)
            v = (xv - mv) / qs
            r = (v + _ROUND_C) - _ROUND_C
            ob[pl.ds(off, _L)] = r * qs + mv

        hout[b] = pltpu.async_copy(ob, out_hbm.at[sl(c)], osems[b])
        hin = nxt
    for h in hout:
        if h is not None:
            h.wait()


@jax.jit
def kernel(inputs, scale, mean, scale_table):
    B, H, W = inputs.shape
    n = B * H * W
    x = inputs.reshape(n)
    s = scale.reshape(n)
    m = mean.reshape(n)

    tab = jnp.concatenate(
        [scale_table, jnp.full((_TPAD - 64,), 1e30, jnp.float32)])
    t0 = scale_table[0]
    t63 = scale_table[63]
    dlog2 = (jnp.log2(t63) - jnp.log2(t0)) * jnp.float32(1.0 / 63.0)
    a = jnp.float32(1.0 / 8388608.0) / dlog2
    b = -(jnp.float32(126.9569643) + jnp.log2(t0)) / dlog2
    cst = jnp.concatenate([jnp.full((_L,), a, jnp.float32),
                           jnp.full((_L,), b, jnp.float32)])

    mesh = plsc.VectorSubcoreMesh(core_axis_name="c", subcore_axis_name="s")
    fn = functools.partial(
        pl.kernel,
        mesh=mesh,
        out_type=jax.ShapeDtypeStruct((n,), jnp.float32),
        compiler_params=pltpu.CompilerParams(needs_layout_passes=False),
        scratch_types=[
            pltpu.VMEM((_TPAD,), jnp.float32),
            pltpu.VMEM((2 * _L,), jnp.float32),
            pltpu.VMEM((_CH,), jnp.float32),
            pltpu.VMEM((_CH,), jnp.float32),
            pltpu.VMEM((_CH,), jnp.float32),
            pltpu.VMEM((_CH,), jnp.float32),
            pltpu.VMEM((_CH,), jnp.float32),
            pltpu.VMEM((_CH,), jnp.float32),
            pltpu.VMEM((_CH,), jnp.float32),
            pltpu.VMEM((_CH,), jnp.float32),
            pltpu.SemaphoreType.DMA,
            pltpu.SemaphoreType.DMA,
            pltpu.SemaphoreType.DMA,
            pltpu.SemaphoreType.DMA,
        ],
    )(_sc_body)
    out = fn(x, s, m, tab, cst)
    return out.reshape(B, H, W)
